# Initial kernel scaffold; baseline (speedup 1.0000x reference)
import jax
import jax.numpy as jnp
from jax.experimental import pallas as pl


def kernel(x, edge_index, Wq, bq, Wk, bk, Wv, bv, Wo, bo):
    raise NotImplementedError("write your pallas kernel here")


# speedup vs baseline: 139.3080x; 139.3080x over previous
import functools
import math

import jax
import jax.numpy as jnp
from jax import lax
from jax.experimental import pallas as pl
from jax.experimental.pallas import tpu as pltpu
from jax.experimental.pallas import tpu_sc as plsc

N = 10000
E = 640000
H = 8
HEAD_DIM = 64
NC = 2
NS = 16
NW = NC * NS
EW = E // NW
C = 400
NCHUNK = EW // C
G = C // 16
NROW = N // NS


def _i32full(v):
    return jnp.full((16,), v, jnp.int32)


def _scores_kernel(p48, xa5, row, col, scores_out, wmax_out,
                   xa_v, row_v, col_v, pbuf, sc_v, wmax_v, sem):
    wid = lax.axis_index("s") * NC + lax.axis_index("c")
    pltpu.sync_copy(xa5, xa_v)
    neg = jnp.full((16,), -3.0e38, jnp.float32)
    lane0 = lax.iota(jnp.int32, 16)

    def chunk_body(k, rmax):
        base = wid * EW + k * C
        pltpu.sync_copy(row.at[pl.ds(base, C)], row_v)
        pltpu.sync_copy(col.at[pl.ds(base, C)], col_v)
        pltpu.async_copy(p48.at[row_v], pbuf, sem).wait()

        def group_body(g, rmax):
            ecol = col_v[pl.ds(g * 16, 16)]
            lane = lane0 + g * 16
            xac = [plsc.load_gather(xa_v, [ecol, _i32full(j)]) for j in range(5)]
            out = []
            for h in range(H):
                acc = plsc.load_gather(pbuf, [lane, _i32full(h * 6)]) * xac[0]
                for j in range(1, 5):
                    acc = acc + plsc.load_gather(pbuf, [lane, _i32full(h * 6 + j)]) * xac[j]
                sc_v[h, pl.ds(g * 16, 16)] = acc
                out.append(jnp.maximum(rmax[h], acc))
            return tuple(out)

        rmax = lax.fori_loop(0, G, group_body, rmax)
        pltpu.sync_copy(sc_v, scores_out.at[wid, k])
        return rmax

    rmax = lax.fori_loop(0, NCHUNK, chunk_body, tuple(neg for _ in range(H)))
    for h in range(H):
        wmax_v[h, :] = rmax[h]
    pltpu.sync_copy(wmax_v, wmax_out.at[wid])


def _aggregate_kernel(scores, m16, xa16, row, col, zeros48, s_out,
                      row_v, col_v, sc_v, obuf, xbuf, m_v, s_shared, sem):
    cid = lax.axis_index("c")
    sid = lax.axis_index("s")
    wid = sid * NC + cid
    pltpu.sync_copy(m16, m_v)
    r0 = sid * NROW
    pltpu.sync_copy(zeros48.at[pl.ds(r0, NROW)], s_shared.at[pl.ds(r0, NROW)])
    plsc.subcore_barrier()
    lane0 = lax.iota(jnp.int32, 16)

    def chunk_body(k, carry):
        base = wid * EW + k * C
        pltpu.sync_copy(row.at[pl.ds(base, C)], row_v)
        pltpu.sync_copy(col.at[pl.ds(base, C)], col_v)
        pltpu.sync_copy(scores.at[wid, k], sc_v)
        pltpu.async_copy(xa16.at[row_v], xbuf, sem).wait()

        def group_body(g, carry):
            lane = lane0 + g * 16
            xar = [plsc.load_gather(xbuf, [lane, _i32full(j)]) for j in range(5)]
            for h in range(H):
                w = jnp.exp(sc_v[h, pl.ds(g * 16, 16)] - m_v[h, :])
                for j in range(5):
                    plsc.store_scatter(obuf, [lane, _i32full(h * 6 + j)], w * xar[j])
            return carry

        lax.fori_loop(0, G, group_body, 0)
        pltpu.sync_copy(obuf, s_shared.at[col_v], add=True)
        return carry

    lax.fori_loop(0, NCHUNK, chunk_body, 0)
    plsc.subcore_barrier()
    pltpu.sync_copy(s_shared.at[pl.ds(r0, NROW)], s_out.at[cid, pl.ds(r0, NROW)])


_mesh = plsc.VectorSubcoreMesh(core_axis_name="c", subcore_axis_name="s")

_scores_call = functools.partial(
    pl.kernel,
    out_type=[jax.ShapeDtypeStruct((NW, NCHUNK, H, C), jnp.float32),
              jax.ShapeDtypeStruct((NW, H, 16), jnp.float32)],
    mesh=_mesh,
    scratch_types=[
        pltpu.VMEM((N, 5), jnp.float32),
        pltpu.VMEM((C,), jnp.int32),
        pltpu.VMEM((C,), jnp.int32),
        pltpu.VMEM((C, 48), jnp.float32),
        pltpu.VMEM((H, C), jnp.float32),
        pltpu.VMEM((H, 16), jnp.float32),
        pltpu.SemaphoreType.DMA,
    ],
    compiler_params=pltpu.CompilerParams(needs_layout_passes=False, use_tc_tiling_on_sc=False),
)(_scores_kernel)

_aggregate_call = functools.partial(
    pl.kernel,
    out_type=jax.ShapeDtypeStruct((NC, N, 48), jnp.float32),
    mesh=_mesh,
    scratch_types=[
        pltpu.VMEM((C,), jnp.int32),
        pltpu.VMEM((C,), jnp.int32),
        pltpu.VMEM((H, C), jnp.float32),
        pltpu.VMEM((C, 48), jnp.float32),
        pltpu.VMEM((C, 16), jnp.float32),
        pltpu.VMEM((H, 16), jnp.float32),
        pltpu.VMEM_SHARED((N, 48), jnp.float32),
        pltpu.SemaphoreType.DMA,
    ],
    compiler_params=pltpu.CompilerParams(needs_layout_passes=False, use_tc_tiling_on_sc=False),
)(_aggregate_kernel)


def kernel(x, edge_index, Wq, bq, Wk, bk, Wv, bv, Wo, bo):
    f32 = jnp.float32
    hp = jax.lax.Precision.HIGHEST
    x = x.astype(f32)
    row = edge_index[0].astype(jnp.int32)
    col = edge_index[1].astype(jnp.int32)

    Wqh = Wq.reshape(H, HEAD_DIM, 4)
    Wkh = Wk.reshape(H, HEAD_DIM, 4)
    Wvh = Wv.reshape(H, HEAD_DIM, 4)
    bqh = bq.reshape(H, HEAD_DIM)
    bkh = bk.reshape(H, HEAD_DIM)
    bvh = bv.reshape(H, HEAD_DIM)
    A = jnp.einsum('hdi,hdj->hij', Wqh, Wkh, precision=hp)
    u = jnp.einsum('hd,hdj->hj', bqh, Wkh, precision=hp)
    v = jnp.einsum('hdi,hd->hi', Wqh, bkh, precision=hp)
    cterm = jnp.sum(bqh * bkh, axis=1)

    inv = 1.0 / math.sqrt(float(HEAD_DIM))
    P4 = (jnp.einsum('ni,hij->nhj', x, A, precision=hp) + u[None]) * inv
    P1 = (jnp.einsum('ni,hi->nh', x, v, precision=hp) + cterm[None]) * inv
    P6 = jnp.concatenate(
        [P4, P1[:, :, None], jnp.zeros((N, H, 1), f32)], axis=2)
    p48 = P6.reshape(N, 48)
    xa5 = jnp.concatenate([x, jnp.ones((N, 1), f32)], axis=1)
    xa16 = jnp.concatenate([xa5, jnp.zeros((N, 11), f32)], axis=1)

    scores, wmax = _scores_call(p48, xa5, row, col)
    m = jnp.max(wmax, axis=(0, 2))
    m16 = jnp.broadcast_to(m[:, None], (H, 16))

    zeros48 = jnp.zeros((N, 48), f32)
    s_parts = _aggregate_call(scores, m16, xa16, row, col, zeros48)

    S = (s_parts[0] + s_parts[1]).reshape(N, H, 6)[:, :, :5]
    Z = jnp.sum(S[:, :, 4], axis=0)
    Sn = S / Z[None, :, None]
    Wvb = jnp.concatenate([Wvh, bvh[:, :, None]], axis=2)
    Woh = jnp.transpose(Wo.reshape(4, H, HEAD_DIM), (1, 0, 2))
    B = jnp.einsum('had,hdj->haj', Woh, Wvb, precision=hp)
    out = jnp.einsum('nhj,haj->na', Sn, B, precision=hp) + bo[None] + x
    return out
